# Initial kernel scaffold; baseline (speedup 1.0000x reference)
import jax
import jax.numpy as jnp
from jax.experimental import pallas as pl


def kernel(edge_index, gs0_edge_index, gs0_eids, gs1_edge_index, gs1_eids, metapath_emb, task_emb, E_table, rgcn0_Wroot, rgcn0_Wrel, rgcn0_b, rgcn1_Wroot, rgcn1_Wrel, rgcn1_b, mp0_Wroot, mp0_Wrel, mp0_b, mp1_Wroot, mp1_Wrel, mp1_b, mp2_Wroot, mp2_Wrel, mp2_b, mp3_Wroot, mp3_Wrel, mp3_b, q_W, q_b, pred_W, pred_b, sa_W1, sa_b1, sa_W2, er_W, er_b):
    raise NotImplementedError("write your pallas kernel here")


# speedup vs baseline: 2.4723x; 2.4723x over previous
import functools

import jax
import jax.numpy as jnp
from jax import lax
from jax.experimental import pallas as pl
from jax.experimental.pallas import tpu as pltpu
from jax.experimental.pallas import tpu_sc as plsc

NE = 10000
D = 128
NREG = 5000
NP = 10240
ACC = 10240
NC = 2
NS = 16
NW = NC * NS
CH = 128
RPT = ACC // NS
CPT = RPT // CH

_f32 = jnp.float32
_i32 = jnp.int32


@functools.lru_cache(maxsize=None)
def _sc_agg(K: int):
    mesh = plsc.VectorSubcoreMesh(core_axis_name="c", subcore_axis_name="s",
                                  num_cores=NC, num_subcores=NS)

    def body(y, srcH, dstH, out_f, src_v, dst_v, rows_v, acc_f, sem):
        c = lax.axis_index("c")
        s = lax.axis_index("s")
        wid = s * NC + c

        def zrow(k, _):
            rows_v[k // 8, pl.ds((k % 8) * 16, 16)] = jnp.zeros((16,), _f32)
            return _
        lax.fori_loop(0, CH * 8, zrow, None)

        r0 = s * RPT

        def zacc(t, _):
            pltpu.sync_copy(rows_v, acc_f.at[pl.ds(r0 + t * CH, CH)])
            return _
        lax.fori_loop(0, CPT, zacc, None)

        plsc.subcore_barrier()

        def step(j, _):
            pltpu.sync_copy(srcH.at[wid, j], src_v)
            pltpu.sync_copy(dstH.at[wid, j], dst_v)
            pltpu.async_copy(y.at[src_v], rows_v, sem).wait()
            pltpu.sync_copy(rows_v, acc_f.at[dst_v], add=True)
            return _
        lax.fori_loop(0, K, step, None)

        plsc.subcore_barrier()

        o0 = c * ACC + s * RPT

        def out_t(t, _):
            pltpu.sync_copy(acc_f.at[pl.ds(r0 + t * CH, CH)], rows_v)
            pltpu.sync_copy(rows_v, out_f.at[pl.ds(o0 + t * CH, CH)])
            return _
        lax.fori_loop(0, CPT, out_t, None)

    return pl.kernel(
        body,
        out_type=jax.ShapeDtypeStruct((NC * ACC, D), _f32),
        mesh=mesh,
        scratch_types=[
            pltpu.VMEM((CH,), _i32),
            pltpu.VMEM((CH,), _i32),
            pltpu.VMEM((CH, D), _f32),
            pltpu.VMEM_SHARED((ACC, D), _f32),
            pltpu.SemaphoreType.DMA,
        ])


@functools.lru_cache(maxsize=None)
def _sc_deg(K: int):
    mesh = plsc.VectorSubcoreMesh(core_axis_name="c", subcore_axis_name="s",
                                  num_cores=NC, num_subcores=NS)

    def body(dstH, out_f, dst_v, rows_v, acc_f):
        c = lax.axis_index("c")
        s = lax.axis_index("s")
        wid = s * NC + c

        def zrow(k, _):
            rows_v[k // 8, pl.ds((k % 8) * 16, 16)] = jnp.zeros((16,), _f32)
            return _
        lax.fori_loop(0, CH * 8, zrow, None)

        r0 = s * RPT

        def zacc(t, _):
            pltpu.sync_copy(rows_v, acc_f.at[pl.ds(r0 + t * CH, CH)])
            return _
        lax.fori_loop(0, CPT, zacc, None)

        def frow(k, _):
            rows_v[k // 8, pl.ds((k % 8) * 16, 16)] = jnp.ones((16,), _f32)
            return _
        lax.fori_loop(0, CH * 8, frow, None)

        plsc.subcore_barrier()

        def step(j, _):
            pltpu.sync_copy(dstH.at[wid, j], dst_v)
            pltpu.sync_copy(rows_v, acc_f.at[dst_v], add=True)
            return _
        lax.fori_loop(0, K, step, None)

        plsc.subcore_barrier()

        o0 = c * ACC + s * RPT

        def out_t(t, _):
            pltpu.sync_copy(acc_f.at[pl.ds(r0 + t * CH, CH)], rows_v)
            pltpu.sync_copy(rows_v, out_f.at[pl.ds(o0 + t * CH, CH)])
            return _
        lax.fori_loop(0, CPT, out_t, None)

    return pl.kernel(
        body,
        out_type=jax.ShapeDtypeStruct((NC * ACC, D), _f32),
        mesh=mesh,
        scratch_types=[
            pltpu.VMEM((CH,), _i32),
            pltpu.VMEM((CH, D), _f32),
            pltpu.VMEM_SHARED((ACC, D), _f32),
        ])


@functools.lru_cache(maxsize=None)
def _sc_gather():
    CG = 64
    KG = NP // (NW * CG)
    mesh = plsc.VectorSubcoreMesh(core_axis_name="c", subcore_axis_name="s",
                                  num_cores=NC, num_subcores=NS)

    def body(table, idxH, out, idx_v, rows_v, sem):
        c = lax.axis_index("c")
        s = lax.axis_index("s")
        wid = s * NC + c
        base = wid * (KG * CG)

        def step(j, _):
            pltpu.sync_copy(idxH.at[wid, j], idx_v)
            pltpu.async_copy(table.at[idx_v], rows_v, sem).wait()
            pltpu.sync_copy(rows_v, out.at[pl.ds(base + j * CG, CG)])
            return _
        lax.fori_loop(0, KG, step, None)

    return pl.kernel(
        body,
        out_type=jax.ShapeDtypeStruct((NP, D), _f32),
        mesh=mesh,
        scratch_types=[
            pltpu.VMEM((CG,), _i32),
            pltpu.VMEM((CG, D), _f32),
            pltpu.SemaphoreType.DMA,
        ])


_BLK = 2048


def _mm2(x, wrel, wroot):
    def body(x_ref, wa, wb, oa, ob):
        xv = x_ref[...]
        oa[...] = jnp.dot(xv, wa[...], preferred_element_type=_f32)
        ob[...] = jnp.dot(xv, wb[...], preferred_element_type=_f32)

    g = NP // _BLK
    return pl.pallas_call(
        body,
        grid=(g,),
        in_specs=[
            pl.BlockSpec((_BLK, D), lambda i: (i, 0)),
            pl.BlockSpec((D, D), lambda i: (0, 0)),
            pl.BlockSpec((D, D), lambda i: (0, 0)),
        ],
        out_specs=[
            pl.BlockSpec((_BLK, D), lambda i: (i, 0)),
            pl.BlockSpec((_BLK, D), lambda i: (i, 0)),
        ],
        out_shape=[jax.ShapeDtypeStruct((NP, D), _f32)] * 2,
    )(x, wrel, wroot)


def _combine(root, feat, deg8, b, act):
    def body(r_ref, f0_ref, f1_ref, d0_ref, d1_ref, b_ref, o_ref):
        deg = jnp.maximum(d0_ref[:, 0:1] + d1_ref[:, 0:1], 1.0)
        o_ref[...] = act(r_ref[...] + (f0_ref[...] + f1_ref[...]) / deg
                         + b_ref[...])

    g = NP // _BLK
    nb = ACC // _BLK
    return pl.pallas_call(
        body,
        grid=(g,),
        in_specs=[
            pl.BlockSpec((_BLK, D), lambda i: (i, 0)),
            pl.BlockSpec((_BLK, D), lambda i: (i, 0)),
            pl.BlockSpec((_BLK, D), lambda i: (i + nb, 0)),
            pl.BlockSpec((_BLK, 8), lambda i: (i, 0)),
            pl.BlockSpec((_BLK, 8), lambda i: (i + nb, 0)),
            pl.BlockSpec((1, D), lambda i: (0, 0)),
        ],
        out_specs=pl.BlockSpec((_BLK, D), lambda i: (i, 0)),
        out_shape=jax.ShapeDtypeStruct((NP, D), _f32),
    )(root, feat, feat, deg8, deg8, b)


def _head(f0, f1, e2, task, mp_emb, q_W, q_b, pred_W, pred_b,
          sa_W1, sa_b1, sa_W2, er_Wa, er_Wb, er_b):
    rs = 1.0 / (D ** 0.5)

    def body(f0r, f1r, e2r, tkr, mpr, qwr, qbr, pwr, pbr,
             w1r, b1r, w2r, ear, ebr, ebi, out):
        q = jnp.dot(mpr[...], qwr[...], preferred_element_type=_f32) + qbr[...]
        f0v = f0r[...]
        f1v = f1r[...]
        s0 = jnp.sum(f0v * q[0:1, :], axis=1, keepdims=True) * rs
        s1 = jnp.sum(f1v * q[1:2, :], axis=1, keepdims=True) * rs
        m = jnp.maximum(s0, s1)
        e0 = jnp.exp(s0 - m)
        e1 = jnp.exp(s1 - m)
        z = e0 + e1
        h = (e0 / z) * f0v + (e1 / z) * f1v
        ereg = jnp.dot(h, pwr[...], preferred_element_type=_f32) + pbr[...]
        tk = tkr[...]
        t0 = jnp.dot(jnp.tanh(
            jnp.dot(ereg, w1r[...], preferred_element_type=_f32) + b1r[...]),
            w2r[...], preferred_element_type=_f32)
        t1 = jnp.dot(jnp.tanh(
            jnp.dot(tk, w1r[...], preferred_element_type=_f32) + b1r[...]),
            w2r[...], preferred_element_type=_f32)
        w0 = jnp.sum(t0, axis=0, keepdims=True) * (1.0 / NREG)
        w1 = jnp.sum(t1, axis=0, keepdims=True) * (1.0 / NREG)
        mw = jnp.maximum(w0, w1)
        g0 = jnp.exp(w0 - mw)
        g1 = jnp.exp(w1 - mw)
        gz = g0 + g1
        ereg2 = (g0 / gz) * ereg + (g1 / gz) * tk
        out[...] = (jnp.dot(ereg2, ear[...], preferred_element_type=_f32)
                    + jnp.dot(e2r[...], ebr[...],
                              preferred_element_type=_f32)
                    + ebi[...])

    row_spec = pl.BlockSpec((NREG, D), lambda: (0, 0))
    return pl.pallas_call(
        body,
        in_specs=[
            row_spec, row_spec, row_spec, row_spec,
            pl.BlockSpec((2, D), lambda: (0, 0)),
            pl.BlockSpec((D, D), lambda: (0, 0)),
            pl.BlockSpec((1, D), lambda: (0, 0)),
            pl.BlockSpec((D, D), lambda: (0, 0)),
            pl.BlockSpec((1, D), lambda: (0, 0)),
            pl.BlockSpec((D, D), lambda: (0, 0)),
            pl.BlockSpec((1, D), lambda: (0, 0)),
            pl.BlockSpec((D, 1), lambda: (0, 0)),
            pl.BlockSpec((D, 1), lambda: (0, 0)),
            pl.BlockSpec((D, 1), lambda: (0, 0)),
            pl.BlockSpec((1, 1), lambda: (0, 0)),
        ],
        out_specs=pl.BlockSpec((NREG, 1), lambda: (0, 0)),
        out_shape=jax.ShapeDtypeStruct((NREG, 1), _f32),
    )(f0, f1, e2, task, mp_emb, q_W, q_b, pred_W, pred_b,
      sa_W1, sa_b1, sa_W2, er_Wa, er_Wb, er_b)


def _pad_edges(ei, K):
    total = NW * K * CH
    src = ei[0].astype(_i32)
    dst = ei[1].astype(_i32)
    e = src.shape[0]
    src = jnp.concatenate([src, jnp.zeros((total - e,), _i32)])
    dst = jnp.concatenate([dst, jnp.full((total - e,), NE, _i32)])
    return src.reshape(NW, K, CH), dst.reshape(NW, K, CH)


def _pad_idx(idx):
    CG = 64
    KG = NP // (NW * CG)
    idx = jnp.concatenate(
        [idx.astype(_i32), jnp.zeros((NP - idx.shape[0],), _i32)])
    return idx.reshape(NW, KG, CG)


def _rgcn_pair(x, ei_pad, K, W0r, W0l, b0, W1r, W1l, b1, act):
    srcH, dstH = ei_pad
    agg = _sc_agg(K)
    deg8 = _sc_deg(K)(dstH)[:, :8]

    y, root = _mm2(x, W0l, W0r)
    feat = agg(y, srcH, dstH)
    x1 = _combine(root, feat, deg8, b0.reshape(1, D), act)

    y, root = _mm2(x1, W1l, W1r)
    feat = agg(y, srcH, dstH)
    x2 = _combine(root, feat, deg8, b1.reshape(1, D), act)
    return x2


def kernel(edge_index, gs0_edge_index, gs0_eids, gs1_edge_index, gs1_eids,
           metapath_emb, task_emb, E_table,
           rgcn0_Wroot, rgcn0_Wrel, rgcn0_b, rgcn1_Wroot, rgcn1_Wrel, rgcn1_b,
           mp0_Wroot, mp0_Wrel, mp0_b, mp1_Wroot, mp1_Wrel, mp1_b,
           mp2_Wroot, mp2_Wrel, mp2_b, mp3_Wroot, mp3_Wrel, mp3_b,
           q_W, q_b, pred_W, pred_b, sa_W1, sa_b1, sa_W2, er_W, er_b):
    KG_ = -(-edge_index.shape[1] // (NW * CH))
    KM_ = -(-gs0_edge_index.shape[1] // (NW * CH))

    eig = _pad_edges(edge_index, KG_)
    ei0 = _pad_edges(gs0_edge_index, KM_)
    ei1 = _pad_edges(gs1_edge_index, KM_)

    x = jnp.concatenate([E_table, jnp.zeros((NP - NE, D), _f32)], axis=0)

    E2 = _rgcn_pair(x, eig, KG_, rgcn0_Wroot, rgcn0_Wrel, rgcn0_b,
                    rgcn1_Wroot, rgcn1_Wrel, rgcn1_b, jnp.tanh)

    gather = _sc_gather()
    relu = jax.nn.relu
    f0 = gather(E2, _pad_idx(gs0_eids))
    f0 = _rgcn_pair(f0, ei0, KM_, mp0_Wroot, mp0_Wrel, mp0_b,
                    mp1_Wroot, mp1_Wrel, mp1_b, relu)
    f1 = gather(E2, _pad_idx(gs1_eids))
    f1 = _rgcn_pair(f1, ei1, KM_, mp2_Wroot, mp2_Wrel, mp2_b,
                    mp3_Wroot, mp3_Wrel, mp3_b, relu)

    pred = _head(f0[:NREG], f1[:NREG], E2[:NREG], task_emb,
                 metapath_emb, q_W, q_b.reshape(1, D),
                 pred_W, pred_b.reshape(1, D),
                 sa_W1, sa_b1.reshape(1, D), sa_W2,
                 er_W[:D], er_W[D:], er_b.reshape(1, 1))
    return pred
